# Initial kernel scaffold; baseline (speedup 1.0000x reference)
import jax
import jax.numpy as jnp
from jax.experimental import pallas as pl


def kernel(user_emb, item_emb, edge_weight, edge_index, users, pos, neg):
    raise NotImplementedError("write your pallas kernel here")


# speedup vs baseline: 2.3034x; 2.3034x over previous
import functools

import jax
import jax.numpy as jnp
from jax import lax
from jax.experimental import pallas as pl
from jax.experimental.pallas import tpu as pltpu
from jax.experimental.pallas import tpu_sc as plsc

NU = 20000
NI = 30000
NN = NU + NI
D = 64
E = 800000
B = 2048
HALF = NN // 2
HP = 25088
NP = 2 * HP
ACC_PT = 1569
ACC = 16 * ACC_PT
DUMMY = HP
NS = 16
NC = 2
EPT = 51200
EPAD = NS * EPT
BL = 1024
NBLK = EPT // BL
CH = 128
NCH = BL // CH
NRB = 3
OUT_PT = HP // NS
BPT = B // (NS * NC)


def _mesh():
  return plsc.VectorSubcoreMesh(core_axis_name="c", subcore_axis_name="s")


def _layer(table, srcv, dstv, wv, zrows):
  @functools.partial(
      pl.kernel,
      out_type=jax.ShapeDtypeStruct((NP, D), jnp.float32),
      mesh=_mesh(),
      compiler_params=pltpu.CompilerParams(use_tc_tiling_on_sc=False),
      scratch_types=[
          pltpu.VMEM((BL,), jnp.int32),
          pltpu.VMEM((BL,), jnp.float32),
          pltpu.VMEM((NCH, CH), jnp.int32),
          pltpu.VMEM((NRB, CH, D), jnp.float32),
          pltpu.VMEM_SHARED((ACC, D), jnp.float32),
          pltpu.SemaphoreType.DMA,
          pltpu.SemaphoreType.DMA,
      ],
  )
  def body(table_h, src_h, dst_h, w_h, z_h, out_h,
           sbuf, wbuf, dlbuf, rows, acc, gsem, ssem):
    core = lax.axis_index("c")
    sub = lax.axis_index("s")
    lo = core * HALF

    pltpu.sync_copy(z_h, acc.at[pl.ds(sub * ACC_PT, ACC_PT)])
    plsc.subcore_barrier()

    iota16 = lax.iota(jnp.int32, 16)

    def gather(j, rb):
      return pltpu.make_async_copy(
          table_h.at[sbuf.at[pl.ds(j * CH, CH)]], rows.at[rb], gsem)

    def scatter(j, rb):
      return pltpu.make_async_copy(rows.at[rb], acc.at[dlbuf.at[j]], ssem)

    def block_body(b, carry):
      off = pl.multiple_of(sub * EPT + b * BL, BL)
      pltpu.sync_copy(src_h.at[pl.ds(off, BL)], sbuf)
      pltpu.sync_copy(w_h.at[pl.ds(off, BL)], wbuf)
      for j in range(NCH):
        pltpu.sync_copy(dst_h.at[pl.ds(off + j * CH, CH)], dlbuf.at[j])

      def remap(g, c):
        goff = pl.multiple_of(g * 16, 16)
        s = sbuf[pl.ds(goff, 16)]
        sbuf[pl.ds(goff, 16)] = jnp.where(s >= HALF, s + (HP - HALF), s)
        j = g >> 3
        po = pl.multiple_of((g & 7) * 16, 16)
        dl = dlbuf[j, pl.ds(po, 16)] - lo
        inr = (dl >= 0) & (dl < HALF)
        dlbuf[j, pl.ds(po, 16)] = jnp.where(inr, dl, DUMMY + iota16)
        return c
      lax.fori_loop(0, BL // 16, remap, 0)

      gather(0, 0).start()
      gather(1, 1).start()
      for j in range(NCH):
        rb = j % NRB
        gather(j, rb).wait()

        def mul16(g, c):
          goff = pl.multiple_of(j * CH + g * 16, 16)
          wv = wbuf[pl.ds(goff, 16)]
          for u in range(16):
            e = (g << 4) + u
            wsc = wv[u]
            for k in range(4):
              sl = pl.ds(k * 16, 16)
              rows[rb, e, sl] = rows[rb, e, sl] * wsc
          return c
        lax.fori_loop(0, CH // 16, mul16, 0)

        scatter(j, rb).start(add=True)
        if j + 2 < NCH:
          if j >= 1:
            scatter(j - 1, (j - 1) % NRB).wait()
          gather(j + 2, (j + 2) % NRB).start()
      for j in range(max(NCH - 3, 0), NCH):
        scatter(j, j % NRB).wait()
      return carry

    lax.fori_loop(0, NBLK, block_body, 0)

    plsc.subcore_barrier()
    pltpu.sync_copy(acc.at[pl.ds(sub * OUT_PT, OUT_PT)],
                    out_h.at[pl.ds(core * HP + sub * OUT_PT, OUT_PT)])

  return body(table, srcv, dstv, wv, zrows)


def _batch_stage(e0, e1, e2, e3, users, pos, neg):
  @functools.partial(
      pl.kernel,
      out_type=(
          jax.ShapeDtypeStruct((B,), jnp.float32),
          jax.ShapeDtypeStruct((B,), jnp.float32),
          jax.ShapeDtypeStruct((NS * NC, 16), jnp.float32),
      ),
      mesh=_mesh(),
      compiler_params=pltpu.CompilerParams(
          use_tc_tiling_on_sc=False, needs_layout_passes=False),
      scratch_types=[
          pltpu.VMEM((BPT,), jnp.int32),
          pltpu.VMEM((BPT,), jnp.int32),
          pltpu.VMEM((BPT,), jnp.int32),
          [pltpu.VMEM((BPT, D), jnp.float32)] * 4,
          [pltpu.VMEM((BPT, D), jnp.float32)] * 4,
          [pltpu.VMEM((BPT, D), jnp.float32)] * 4,
          pltpu.VMEM((BPT,), jnp.float32),
          pltpu.VMEM((BPT,), jnp.float32),
          pltpu.VMEM((16,), jnp.float32),
          pltpu.SemaphoreType.DMA,
      ],
  )
  def body(e0_h, e1_h, e2_h, e3_h, users_h, pos_h, neg_h,
           ps_h, ns_h, regp_h,
           ubuf, pibuf, nibuf, ru, rp, rn, psbuf, nsbuf, regv, sem):
    core = lax.axis_index("c")
    sub = lax.axis_index("s")
    wid = sub * NC + core
    boff = pl.multiple_of(wid * BPT, BPT)

    pltpu.sync_copy(users_h.at[pl.ds(boff, BPT)], ubuf)
    pltpu.sync_copy(pos_h.at[pl.ds(boff, BPT)], pibuf)
    pltpu.sync_copy(neg_h.at[pl.ds(boff, BPT)], nibuf)

    def remap(g, c):
      goff = pl.multiple_of(g * 16, 16)
      p = pibuf[pl.ds(goff, 16)] + NU
      pibuf[pl.ds(goff, 16)] = jnp.where(p >= HALF, p + (HP - HALF), p)
      n = nibuf[pl.ds(goff, 16)] + NU
      nibuf[pl.ds(goff, 16)] = jnp.where(n >= HALF, n + (HP - HALF), n)
      return c
    lax.fori_loop(0, BPT // 16, remap, 0)

    tabs = (e0_h, e1_h, e2_h, e3_h)
    copies = []
    for l in range(4):
      copies.append(pltpu.make_async_copy(tabs[l].at[ubuf], ru[l], sem))
      copies.append(pltpu.make_async_copy(tabs[l].at[pibuf], rp[l], sem))
      copies.append(pltpu.make_async_copy(tabs[l].at[nibuf], rn[l], sem))
    for c in copies:
      c.start()
    for c in copies:
      c.wait()

    zero16 = jnp.zeros((16,), jnp.float32)
    iota16 = lax.iota(jnp.int32, 16)

    def rowgrp(g, reg):
      goff = pl.multiple_of(g * 16, 16)
      psvec = zero16
      nsvec = zero16
      for u in range(16):
        e = goff + u
        psv = zero16
        nsv = zero16
        for k in range(4):
          sl = pl.ds(k * 16, 16)
          u0 = ru[0][e, sl]
          p0 = rp[0][e, sl]
          n0 = rn[0][e, sl]
          ue = (u0 + ru[1][e, sl] + ru[2][e, sl] + ru[3][e, sl]) * 0.25
          pe = (p0 + rp[1][e, sl] + rp[2][e, sl] + rp[3][e, sl]) * 0.25
          ne = (n0 + rn[1][e, sl] + rn[2][e, sl] + rn[3][e, sl]) * 0.25
          reg = reg + u0 * u0 + p0 * p0 + n0 * n0
          psv = psv + ue * pe
          nsv = nsv + ue * ne
        lane = iota16 == u
        psvec = jnp.where(lane, jnp.sum(psv), psvec)
        nsvec = jnp.where(lane, jnp.sum(nsv), nsvec)
      psbuf[pl.ds(goff, 16)] = psvec
      nsbuf[pl.ds(goff, 16)] = nsvec
      return reg

    reg = lax.fori_loop(0, BPT // 16, rowgrp, zero16)
    regv[...] = reg

    pltpu.sync_copy(psbuf, ps_h.at[pl.ds(boff, BPT)])
    pltpu.sync_copy(nsbuf, ns_h.at[pl.ds(boff, BPT)])
    pltpu.sync_copy(regv, regp_h.at[wid])

  return body(e0, e1, e2, e3, users, pos, neg)


def _loss_stage(ps2, ns2, regp):
  def tbody(ps_ref, ns_ref, regp_ref, loss_ref, reg_ref):
    x = ns_ref[...] - ps_ref[...]
    sp = jnp.maximum(x, 0.0) + jnp.log1p(jnp.exp(-jnp.abs(x)))
    loss_ref[0, 0] = jnp.sum(sp) / B
    reg_ref[0, 0] = jnp.sum(regp_ref[...]) / (2.0 * B)

  return pl.pallas_call(
      tbody,
      out_shape=(
          jax.ShapeDtypeStruct((1, 1), jnp.float32),
          jax.ShapeDtypeStruct((1, 1), jnp.float32),
      ),
      out_specs=(
          pl.BlockSpec(memory_space=pltpu.SMEM),
          pl.BlockSpec(memory_space=pltpu.SMEM),
      ),
  )(ps2, ns2, regp)


def kernel(user_emb, item_emb, edge_weight, edge_index, users, pos, neg):
  src = edge_index[0].astype(jnp.int32)
  dst = edge_index[1].astype(jnp.int32)
  w = edge_weight.astype(jnp.float32)
  pad = EPAD - E
  src = jnp.concatenate([src, jnp.zeros((pad,), jnp.int32)])
  dst = jnp.concatenate([dst, jnp.full((pad,), NN, jnp.int32)])
  w = jnp.concatenate([w, jnp.zeros((pad,), jnp.float32)])

  zpad = jnp.zeros((HP - HALF, D), jnp.float32)
  e0 = jnp.concatenate(
      [user_emb, item_emb[:HALF - NU], zpad, item_emb[HALF - NU:], zpad],
      axis=0)
  zrows = jnp.zeros((ACC_PT, D), jnp.float32)

  e1 = _layer(e0, src, dst, w, zrows)
  e2 = _layer(e1, src, dst, w, zrows)
  e3 = _layer(e2, src, dst, w, zrows)

  ps, ns, regp = _batch_stage(e0, e1, e2, e3,
                              users.astype(jnp.int32),
                              pos.astype(jnp.int32),
                              neg.astype(jnp.int32))
  loss2, reg2 = _loss_stage(ps.reshape(16, 128), ns.reshape(16, 128), regp)
  return (loss2[0, 0], reg2[0, 0])
